# baseline (speedup 1.0000x reference)
import functools
import math

import jax
import jax.numpy as jnp
from jax import lax
from jax.experimental import pallas as pl
from jax.experimental.pallas import tpu as pltpu
from jax.experimental.pallas import tpu_sc as plsc

N = 10000
E = 320000
D = 128
OUT = 40
EPS = 1e-5
BNS = 1.0 / math.sqrt(1.0 + EPS)

NC, NS = 2, 16
NW = NC * NS
CB = 128
NFULL = 80
EPW = NFULL * CB
EPAD = NW * EPW
NPAD = 10240
RPT = NPAD // NS
QCH = RPT // CB


def _make_agg(with_deg: bool):
    out_type = [jax.ShapeDtypeStruct((NC, NPAD, D), jnp.float32)]
    if with_deg:
        out_type.append(jax.ShapeDtypeStruct((NC, NPAD), jnp.float32))

    scratch = [
        pltpu.VMEM((2, CB), jnp.int32),
        pltpu.VMEM((2, CB), jnp.int32),
        pltpu.VMEM((CB, D), jnp.float32),
        pltpu.VMEM((CB, D), jnp.float32),
        pltpu.VMEM((CB,), jnp.float32),
        pltpu.VMEM((CB,), jnp.float32),
        pltpu.VMEM_SHARED((NPAD, D), jnp.float32),
        pltpu.VMEM_SHARED((NPAD,), jnp.float32),
        pltpu.SemaphoreType.DMA,
        pltpu.SemaphoreType.DMA,
    ]

    def body(h_hbm, edges_hbm, z_hbm, one_hbm, *refs):
        if with_deg:
            aggp, degp = refs[0], refs[1]
            rest = refs[2:]
        else:
            aggp, degp = refs[0], None
            rest = refs[1:]
        (idxA, idxB, rowsA, rowsB, ones_v, dv,
         acc_sh, deg_sh, sem_g, sem_s) = rest

        c = lax.axis_index("c")
        s = lax.axis_index("s")
        w = s * NC + c

        pltpu.sync_copy(z_hbm, rowsA)
        pltpu.sync_copy(one_hbm, ones_v)

        def zero_q(q, carry):
            off = s * RPT + q * CB
            pltpu.sync_copy(rowsA, acc_sh.at[pl.ds(off, CB)])
            if with_deg:
                pltpu.sync_copy(rowsA.at[0], deg_sh.at[pl.ds(off, CB)])
            return carry

        lax.fori_loop(0, QCH, zero_q, 0)
        plsc.subcore_barrier()

        chunk0 = w * NFULL

        def load_idx(g, ibuf):
            pltpu.sync_copy(edges_hbm.at[:, chunk0 + g], ibuf)

        def fire_g(ibuf, rbuf):
            pltpu.async_copy(h_hbm.at[ibuf.at[0]], rbuf, sem_g)

        def drain_g(ibuf, rbuf):
            pltpu.make_async_copy(h_hbm.at[ibuf.at[0]], rbuf, sem_g).wait()

        def fire_s(ibuf, rbuf):
            pltpu.async_copy(rbuf, acc_sh.at[ibuf.at[1]], sem_s, add=True)
            if with_deg:
                pltpu.async_copy(ones_v, deg_sh.at[ibuf.at[1]], sem_s,
                                 add=True)

        def drain_s(ibuf, rbuf):
            pltpu.make_async_copy(rbuf, acc_sh.at[ibuf.at[1]], sem_s).wait()
            if with_deg:
                pltpu.make_async_copy(ones_v, deg_sh.at[ibuf.at[1]],
                                      sem_s).wait()

        load_idx(0, idxA)
        fire_g(idxA, rowsA)
        NIT = NFULL // 2

        def pipe(it, carry):
            g = it * 2
            drain_g(idxA, rowsA)
            fire_s(idxA, rowsA)

            @pl.when(it > 0)
            def _():
                drain_s(idxB, rowsB)

            load_idx(g + 1, idxB)
            fire_g(idxB, rowsB)

            drain_g(idxB, rowsB)
            fire_s(idxB, rowsB)
            drain_s(idxA, rowsA)

            @pl.when(it < NIT - 1)
            def _():
                load_idx(g + 2, idxA)
                fire_g(idxA, rowsA)

            return carry

        lax.fori_loop(0, NIT, pipe, 0)
        drain_s(idxB, rowsB)

        plsc.subcore_barrier()

        def wb_q(q, carry):
            off = s * RPT + q * CB
            pltpu.sync_copy(acc_sh.at[pl.ds(off, CB)], rowsA)
            pltpu.sync_copy(rowsA, aggp.at[c, pl.ds(off, CB)])
            if with_deg:
                pltpu.sync_copy(deg_sh.at[pl.ds(off, CB)], dv)
                pltpu.sync_copy(dv, degp.at[c, pl.ds(off, CB)])
            return carry

        lax.fori_loop(0, QCH, wb_q, 0)

    mesh = plsc.VectorSubcoreMesh(
        core_axis_name="c", subcore_axis_name="s",
        num_cores=NC, num_subcores=NS)
    return pl.kernel(body, out_type=out_type, mesh=mesh,
                     scratch_types=scratch)


_agg_deg = _make_agg(True)
_agg = _make_agg(False)

BR = 1000
GRID = N // BR


def _conv_body(h_ref, p_ref, d_ref, ws_ref, wn_ref, g_ref, b_ref, o_ref):
    dsum = jnp.maximum(d_ref[0] + d_ref[1], 1.0)
    agg = (p_ref[0] + p_ref[1]) / dsum
    rst = (jnp.dot(h_ref[...], ws_ref[...], preferred_element_type=jnp.float32)
           + jnp.dot(agg, wn_ref[...], preferred_element_type=jnp.float32))
    y = rst * (g_ref[0] * BNS) + b_ref[0]
    o_ref[...] = jnp.maximum(y, 0.0)


def _final_body(h_ref, p_ref, d_ref, ws_ref, wn_ref, g_ref, b_ref,
                w0_ref, b0_ref, g0_ref, be0_ref, w1_ref, b1_ref, o_ref):
    dsum = jnp.maximum(d_ref[0] + d_ref[1], 1.0)
    agg = (p_ref[0] + p_ref[1]) / dsum
    rst = (jnp.dot(h_ref[...], ws_ref[...], preferred_element_type=jnp.float32)
           + jnp.dot(agg, wn_ref[...], preferred_element_type=jnp.float32))
    h3 = jnp.maximum(rst * (g_ref[0] * BNS) + b_ref[0], 0.0)
    t = jnp.dot(h3, w0_ref[...], preferred_element_type=jnp.float32) + b0_ref[0]
    t = jnp.maximum(t * (g0_ref[0] * BNS) + be0_ref[0], 0.0)
    o_ref[...] = (jnp.dot(t, w1_ref[...], preferred_element_type=jnp.float32)
                  + b1_ref[0])


_ROWS = pl.BlockSpec((BR, D), lambda i: (i, 0))
_PART = pl.BlockSpec((NC, BR, D), lambda i: (0, i, 0))
_DEG = pl.BlockSpec((NC, BR, 1), lambda i: (0, i, 0))
_MAT = pl.BlockSpec((D, D), lambda i: (0, 0))
_VEC = pl.BlockSpec((1, D), lambda i: (0, 0))

_conv_tc = pl.pallas_call(
    _conv_body,
    grid=(GRID,),
    in_specs=[_ROWS, _PART, _DEG, _MAT, _MAT, _VEC, _VEC],
    out_specs=_ROWS,
    out_shape=jax.ShapeDtypeStruct((N, D), jnp.float32),
)

_final_tc = pl.pallas_call(
    _final_body,
    grid=(GRID,),
    in_specs=[_ROWS, _PART, _DEG, _MAT, _MAT, _VEC, _VEC,
              _MAT, _VEC, _VEC, _VEC,
              pl.BlockSpec((D, OUT), lambda i: (0, 0)),
              pl.BlockSpec((1, OUT), lambda i: (0, 0))],
    out_specs=pl.BlockSpec((BR, OUT), lambda i: (i, 0)),
    out_shape=jax.ShapeDtypeStruct((N, OUT), jnp.float32),
)


def kernel(feat, params, edge_index):
    pad = EPAD - E
    src_p = jnp.concatenate([edge_index[0], jnp.zeros((pad,), jnp.int32)])
    dst_p = jnp.concatenate(
        [edge_index[1],
         N + (jnp.arange(pad, dtype=jnp.int32) % (NPAD - N))])
    edges = jnp.stack([src_p, dst_p]).reshape(2, NW * NFULL, CB)
    zeros = jnp.zeros((CB, D), jnp.float32)
    ones = jnp.ones((CB,), jnp.float32)

    convs = params["convs"]
    c0, c1 = params["cls"][0], params["cls"][1]
    row = lambda v: v.reshape(1, -1)

    h = feat
    degp3 = None
    for i in range(len(convs)):
        p = convs[i]
        if i == 0:
            aggp, degp = _agg_deg(h, edges, zeros, ones)
            degp3 = degp[:, :, None]
        else:
            (aggp,) = _agg(h, edges, zeros, ones)
        if i < len(convs) - 1:
            h = _conv_tc(h, aggp, degp3, p["W_self"], p["W_neigh"],
                         row(p["gamma"]), row(p["beta"]))
        else:
            h = _final_tc(h, aggp, degp3, p["W_self"], p["W_neigh"],
                          row(p["gamma"]), row(p["beta"]),
                          c0["W"], row(c0["b"]), row(c0["gamma"]),
                          row(c0["beta"]), c1["W"], row(c1["b"]))
    return h


# speedup vs baseline: 1.2397x; 1.2397x over previous
import functools
import math

import jax
import jax.numpy as jnp
from jax import lax
from jax.experimental import pallas as pl
from jax.experimental.pallas import tpu as pltpu
from jax.experimental.pallas import tpu_sc as plsc

N = 10000
E = 320000
D = 128
OUT = 40
EPS = 1e-5
BNS = 1.0 / math.sqrt(1.0 + EPS)

NC, NS = 2, 16
NW = NC * NS
CB = 128
NFULL = 80
EPW = NFULL * CB
EPAD = NW * EPW
NPAD = 10240
RPT = NPAD // NS
QCH = RPT // CB


def _make_agg(with_deg: bool):
    out_type = [jax.ShapeDtypeStruct((NC, NPAD, D), jnp.float32)]
    if with_deg:
        out_type.append(jax.ShapeDtypeStruct((NC, NPAD), jnp.float32))

    scratch = [
        pltpu.VMEM((2, CB), jnp.int32),
        pltpu.VMEM((2, CB), jnp.int32),
        pltpu.VMEM((CB, D), jnp.float32),
        pltpu.VMEM((CB, D), jnp.float32),
        pltpu.VMEM((CB,), jnp.float32),
        pltpu.VMEM((CB,), jnp.float32),
        pltpu.VMEM_SHARED((NPAD, D), jnp.float32),
        pltpu.VMEM_SHARED((NPAD,), jnp.float32),
        pltpu.SemaphoreType.DMA,
        pltpu.SemaphoreType.DMA,
    ]

    def body(h_hbm, edges_hbm, z_hbm, one_hbm, *refs):
        if with_deg:
            aggp, degp = refs[0], refs[1]
            rest = refs[2:]
        else:
            aggp, degp = refs[0], None
            rest = refs[1:]
        (idxA, idxB, rowsA, rowsB, ones_v, dv,
         acc_sh, deg_sh, sem_g, sem_s) = rest

        c = lax.axis_index("c")
        s = lax.axis_index("s")
        w = s * NC + c

        pltpu.sync_copy(z_hbm, rowsA)
        pltpu.sync_copy(one_hbm, ones_v)

        def zero_q(q, carry):
            off = s * RPT + q * CB
            pltpu.sync_copy(rowsA, acc_sh.at[pl.ds(off, CB)])
            if with_deg:
                pltpu.sync_copy(rowsA.at[0], deg_sh.at[pl.ds(off, CB)])
            return carry

        lax.fori_loop(0, QCH, zero_q, 0)
        plsc.subcore_barrier()

        chunk0 = w * NFULL

        def load_idx(g, ibuf):
            pltpu.sync_copy(edges_hbm.at[:, chunk0 + g], ibuf)

        def fire_g(ibuf, rbuf):
            pltpu.async_copy(h_hbm.at[ibuf.at[0]], rbuf, sem_g)

        def drain_g(ibuf, rbuf):
            pltpu.make_async_copy(h_hbm.at[ibuf.at[0]], rbuf, sem_g).wait()

        def fire_s(ibuf, rbuf):
            pltpu.async_copy(rbuf, acc_sh.at[ibuf.at[1]], sem_s, add=True)
            if with_deg:
                pltpu.async_copy(ones_v, deg_sh.at[ibuf.at[1]], sem_s,
                                 add=True)

        def drain_s(ibuf, rbuf):
            pltpu.make_async_copy(rbuf, acc_sh.at[ibuf.at[1]], sem_s).wait()
            if with_deg:
                pltpu.make_async_copy(ones_v, deg_sh.at[ibuf.at[1]],
                                      sem_s).wait()

        load_idx(0, idxA)
        fire_g(idxA, rowsA)
        NIT = NFULL // 2

        def pipe(it, carry):
            g = it * 2
            drain_g(idxA, rowsA)
            fire_s(idxA, rowsA)

            @pl.when(it > 0)
            def _():
                drain_s(idxB, rowsB)

            load_idx(g + 1, idxB)
            fire_g(idxB, rowsB)

            drain_g(idxB, rowsB)
            fire_s(idxB, rowsB)
            drain_s(idxA, rowsA)

            @pl.when(it < NIT - 1)
            def _():
                load_idx(g + 2, idxA)
                fire_g(idxA, rowsA)

            return carry

        lax.fori_loop(0, NIT, pipe, 0)
        drain_s(idxB, rowsB)

        plsc.subcore_barrier()

        def wb_q(q, carry):
            off = s * RPT + q * CB
            pltpu.sync_copy(acc_sh.at[pl.ds(off, CB)], rowsA)
            pltpu.sync_copy(rowsA, aggp.at[c, pl.ds(off, CB)])
            if with_deg:
                pltpu.sync_copy(deg_sh.at[pl.ds(off, CB)], dv)
                pltpu.sync_copy(dv, degp.at[c, pl.ds(off, CB)])
            return carry

        lax.fori_loop(0, QCH, wb_q, 0)

    mesh = plsc.VectorSubcoreMesh(
        core_axis_name="c", subcore_axis_name="s",
        num_cores=NC, num_subcores=NS)
    return pl.kernel(body, out_type=out_type, mesh=mesh,
                     scratch_types=scratch)


_agg_deg = _make_agg(True)
_agg = _make_agg(False)

BR = 1000
GRID = N // BR


def _conv_body(h_ref, p_ref, d_ref, ws_ref, wn_ref, g_ref, b_ref, o_ref):
    dsum = jnp.maximum(d_ref[0] + d_ref[1], 1.0)
    agg = (p_ref[0] + p_ref[1]) / dsum
    rst = (jnp.dot(h_ref[...], ws_ref[...], preferred_element_type=jnp.float32)
           + jnp.dot(agg, wn_ref[...], preferred_element_type=jnp.float32))
    y = rst * (g_ref[0] * BNS) + b_ref[0]
    o_ref[...] = jnp.maximum(y, 0.0)


def _final_body(h_ref, p_ref, d_ref, ws_ref, wn_ref, g_ref, b_ref,
                w0_ref, b0_ref, g0_ref, be0_ref, w1_ref, b1_ref, o_ref):
    dsum = jnp.maximum(d_ref[0] + d_ref[1], 1.0)
    agg = (p_ref[0] + p_ref[1]) / dsum
    rst = (jnp.dot(h_ref[...], ws_ref[...], preferred_element_type=jnp.float32)
           + jnp.dot(agg, wn_ref[...], preferred_element_type=jnp.float32))
    h3 = jnp.maximum(rst * (g_ref[0] * BNS) + b_ref[0], 0.0)
    t = jnp.dot(h3, w0_ref[...], preferred_element_type=jnp.float32) + b0_ref[0]
    t = jnp.maximum(t * (g0_ref[0] * BNS) + be0_ref[0], 0.0)
    o_ref[...] = (jnp.dot(t, w1_ref[...], preferred_element_type=jnp.float32)
                  + b1_ref[0])


_ROWS = pl.BlockSpec((BR, D), lambda i: (i, 0))
_PART = pl.BlockSpec((NC, BR, D), lambda i: (0, i, 0))
_DEG = pl.BlockSpec((NC, BR, 1), lambda i: (0, i, 0))
_MAT = pl.BlockSpec((D, D), lambda i: (0, 0))
_VEC = pl.BlockSpec((1, D), lambda i: (0, 0))

_conv_tc = pl.pallas_call(
    _conv_body,
    grid=(GRID,),
    in_specs=[_ROWS, _PART, _DEG, _MAT, _MAT, _VEC, _VEC],
    out_specs=_ROWS,
    out_shape=jax.ShapeDtypeStruct((N, D), jnp.float32),
)

_final_tc = pl.pallas_call(
    _final_body,
    grid=(GRID,),
    in_specs=[_ROWS, _PART, _DEG, _MAT, _MAT, _VEC, _VEC,
              _MAT, _VEC, _VEC, _VEC,
              pl.BlockSpec((D, OUT), lambda i: (0, 0)),
              pl.BlockSpec((1, OUT), lambda i: (0, 0))],
    out_specs=pl.BlockSpec((BR, OUT), lambda i: (i, 0)),
    out_shape=jax.ShapeDtypeStruct((N, OUT), jnp.float32),
)


def kernel(feat, params, edge_index):
    PPW = (EPAD - E) // NW
    real = jnp.stack([edge_index[0], edge_index[1]]).reshape(2, NW, E // NW)
    w_ids = jnp.arange(NW, dtype=jnp.int32)[:, None]
    k_ids = jnp.arange(PPW, dtype=jnp.int32)[None, :]
    pad_dst = N + (w_ids * 15 + k_ids) % (NPAD - N)
    pad_src = jnp.zeros((NW, PPW), jnp.int32)
    padded = jnp.concatenate([real, jnp.stack([pad_src, pad_dst])], axis=2)
    edges = padded.reshape(2, NW * NFULL, CB)
    zeros = jnp.zeros((CB, D), jnp.float32)
    ones = jnp.ones((CB,), jnp.float32)

    convs = params["convs"]
    c0, c1 = params["cls"][0], params["cls"][1]
    row = lambda v: v.reshape(1, -1)

    h = feat
    degp3 = None
    for i in range(len(convs)):
        p = convs[i]
        if i == 0:
            aggp, degp = _agg_deg(h, edges, zeros, ones)
            degp3 = degp[:, :, None]
        else:
            (aggp,) = _agg(h, edges, zeros, ones)
        if i < len(convs) - 1:
            h = _conv_tc(h, aggp, degp3, p["W_self"], p["W_neigh"],
                         row(p["gamma"]), row(p["beta"]))
        else:
            h = _final_tc(h, aggp, degp3, p["W_self"], p["W_neigh"],
                          row(p["gamma"]), row(p["beta"]),
                          c0["W"], row(c0["b"]), row(c0["gamma"]),
                          row(c0["beta"]), c1["W"], row(c1["b"]))
    return h


# speedup vs baseline: 2.5986x; 2.0961x over previous
import functools
import math

import jax
import jax.numpy as jnp
from jax import lax
from jax.experimental import pallas as pl
from jax.experimental.pallas import tpu as pltpu
from jax.experimental.pallas import tpu_sc as plsc

N = 10000
E = 320000
D = 128
OUT = 40
EPS = 1e-5
BNS = 1.0 / math.sqrt(1.0 + EPS)

NC, NS = 2, 16
NW = NC * NS
EPW = E // NW
CB = 128
NFULL = EPW // CB
TAIL = EPW - NFULL * CB
NPAD = 10240
RPT = NPAD // NS
QCH = RPT // CB


def _make_agg(with_deg: bool):
    out_type = [jax.ShapeDtypeStruct((NC, NPAD, D), jnp.float32)]
    if with_deg:
        out_type.append(jax.ShapeDtypeStruct((NC, NPAD), jnp.float32))

    scratch = [
        pltpu.VMEM((CB,), jnp.int32),
        pltpu.VMEM((CB,), jnp.int32),
        pltpu.VMEM((CB,), jnp.int32),
        pltpu.VMEM((CB,), jnp.int32),
        pltpu.VMEM((TAIL,), jnp.int32),
        pltpu.VMEM((TAIL,), jnp.int32),
        pltpu.VMEM((CB, D), jnp.float32),
        pltpu.VMEM((CB, D), jnp.float32),
        pltpu.VMEM((TAIL, D), jnp.float32),
        pltpu.VMEM((CB,), jnp.float32),
        pltpu.VMEM((CB,), jnp.float32),
        pltpu.VMEM_SHARED((NPAD, D), jnp.float32),
        pltpu.VMEM_SHARED((NPAD,), jnp.float32),
        pltpu.SemaphoreType.DMA,
        pltpu.SemaphoreType.DMA,
    ]

    def body(h_hbm, src_hbm, dst_hbm, z_hbm, one_hbm, *refs):
        if with_deg:
            aggp, degp = refs[0], refs[1]
            rest = refs[2:]
        else:
            aggp, degp = refs[0], None
            rest = refs[1:]
        (srcA, srcB, dstA, dstB, s16, d16, rowsA, rowsB, rows16,
         ones_v, dv, acc_sh, deg_sh, sem_g, sem_s) = rest

        c = lax.axis_index("c")
        s = lax.axis_index("s")
        w = s * NC + c

        pltpu.sync_copy(z_hbm, rowsA)
        pltpu.sync_copy(one_hbm, ones_v)

        def zero_q(q, carry):
            off = s * RPT + q * CB
            pltpu.sync_copy(rowsA, acc_sh.at[pl.ds(off, CB)])
            if with_deg:
                pltpu.sync_copy(rowsA.at[0], deg_sh.at[pl.ds(off, CB)])
            return carry

        lax.fori_loop(0, QCH, zero_q, 0)
        plsc.subcore_barrier()

        base0 = w * EPW

        def load_idx(g, sbuf, dbuf):
            b = base0 + g * CB
            pltpu.sync_copy(src_hbm.at[pl.ds(b, CB)], sbuf)
            pltpu.sync_copy(dst_hbm.at[pl.ds(b, CB)], dbuf)

        def fire_g(sbuf, rbuf):
            pltpu.async_copy(h_hbm.at[sbuf], rbuf, sem_g)

        def drain_g(sbuf, rbuf):
            pltpu.make_async_copy(h_hbm.at[sbuf], rbuf, sem_g).wait()

        def fire_s(dbuf, rbuf):
            pltpu.async_copy(rbuf, acc_sh.at[dbuf], sem_s, add=True)
            if with_deg:
                pltpu.async_copy(ones_v, deg_sh.at[dbuf], sem_s, add=True)

        def drain_s(dbuf, rbuf):
            pltpu.make_async_copy(rbuf, acc_sh.at[dbuf], sem_s).wait()
            if with_deg:
                pltpu.make_async_copy(ones_v, deg_sh.at[dbuf], sem_s).wait()

        load_idx(0, srcA, dstA)
        fire_g(srcA, rowsA)
        NIT = NFULL // 2

        def pipe(it, carry):
            g = it * 2
            drain_g(srcA, rowsA)
            fire_s(dstA, rowsA)

            @pl.when(it > 0)
            def _():
                drain_s(dstB, rowsB)

            load_idx(g + 1, srcB, dstB)
            fire_g(srcB, rowsB)

            drain_g(srcB, rowsB)
            fire_s(dstB, rowsB)
            drain_s(dstA, rowsA)

            @pl.when(it < NIT - 1)
            def _():
                load_idx(g + 2, srcA, dstA)
                fire_g(srcA, rowsA)

            return carry

        lax.fori_loop(0, NIT, pipe, 0)
        drain_s(dstB, rowsB)

        bt = base0 + NFULL * CB
        pltpu.sync_copy(src_hbm.at[pl.ds(bt, TAIL)], s16)
        pltpu.sync_copy(dst_hbm.at[pl.ds(bt, TAIL)], d16)
        pltpu.async_copy(h_hbm.at[s16], rows16, sem_g).wait()
        pltpu.sync_copy(rows16, acc_sh.at[d16], add=True)
        if with_deg:
            pltpu.sync_copy(ones_v.at[pl.ds(0, TAIL)], deg_sh.at[d16],
                            add=True)

        plsc.subcore_barrier()

        def wb_q(q, carry):
            off = s * RPT + q * CB
            pltpu.sync_copy(acc_sh.at[pl.ds(off, CB)], rowsA)
            pltpu.sync_copy(rowsA, aggp.at[c, pl.ds(off, CB)])
            if with_deg:
                pltpu.sync_copy(deg_sh.at[pl.ds(off, CB)], dv)
                pltpu.sync_copy(dv, degp.at[c, pl.ds(off, CB)])
            return carry

        lax.fori_loop(0, QCH, wb_q, 0)

    mesh = plsc.VectorSubcoreMesh(
        core_axis_name="c", subcore_axis_name="s",
        num_cores=NC, num_subcores=NS)
    return pl.kernel(body, out_type=out_type, mesh=mesh,
                     scratch_types=scratch)


_agg_deg = _make_agg(True)
_agg = _make_agg(False)

BR = 1000
GRID = N // BR


def _conv_body(h_ref, p_ref, d_ref, ws_ref, wn_ref, g_ref, b_ref, o_ref):
    dsum = jnp.maximum(d_ref[0] + d_ref[1], 1.0)
    agg = (p_ref[0] + p_ref[1]) / dsum
    rst = (jnp.dot(h_ref[...], ws_ref[...], preferred_element_type=jnp.float32)
           + jnp.dot(agg, wn_ref[...], preferred_element_type=jnp.float32))
    y = rst * (g_ref[0] * BNS) + b_ref[0]
    o_ref[...] = jnp.maximum(y, 0.0)


def _final_body(h_ref, p_ref, d_ref, ws_ref, wn_ref, g_ref, b_ref,
                w0_ref, b0_ref, g0_ref, be0_ref, w1_ref, b1_ref, o_ref):
    dsum = jnp.maximum(d_ref[0] + d_ref[1], 1.0)
    agg = (p_ref[0] + p_ref[1]) / dsum
    rst = (jnp.dot(h_ref[...], ws_ref[...], preferred_element_type=jnp.float32)
           + jnp.dot(agg, wn_ref[...], preferred_element_type=jnp.float32))
    h3 = jnp.maximum(rst * (g_ref[0] * BNS) + b_ref[0], 0.0)
    t = jnp.dot(h3, w0_ref[...], preferred_element_type=jnp.float32) + b0_ref[0]
    t = jnp.maximum(t * (g0_ref[0] * BNS) + be0_ref[0], 0.0)
    o_ref[...] = (jnp.dot(t, w1_ref[...], preferred_element_type=jnp.float32)
                  + b1_ref[0])


_ROWS = pl.BlockSpec((BR, D), lambda i: (i, 0))
_PART = pl.BlockSpec((NC, BR, D), lambda i: (0, i, 0))
_DEG = pl.BlockSpec((NC, BR, 1), lambda i: (0, i, 0))
_MAT = pl.BlockSpec((D, D), lambda i: (0, 0))
_VEC = pl.BlockSpec((1, D), lambda i: (0, 0))

_conv_tc = pl.pallas_call(
    _conv_body,
    grid=(GRID,),
    in_specs=[_ROWS, _PART, _DEG, _MAT, _MAT, _VEC, _VEC],
    out_specs=_ROWS,
    out_shape=jax.ShapeDtypeStruct((N, D), jnp.float32),
)

_final_tc = pl.pallas_call(
    _final_body,
    grid=(GRID,),
    in_specs=[_ROWS, _PART, _DEG, _MAT, _MAT, _VEC, _VEC,
              _MAT, _VEC, _VEC, _VEC,
              pl.BlockSpec((D, OUT), lambda i: (0, 0)),
              pl.BlockSpec((1, OUT), lambda i: (0, 0))],
    out_specs=pl.BlockSpec((BR, OUT), lambda i: (i, 0)),
    out_shape=jax.ShapeDtypeStruct((N, OUT), jnp.float32),
)


def kernel(feat, params, edge_index):
    src = edge_index[0]
    dst = edge_index[1]
    zeros = jnp.zeros((CB, D), jnp.float32)
    ones = jnp.ones((CB,), jnp.float32)

    convs = params["convs"]
    c0, c1 = params["cls"][0], params["cls"][1]
    row = lambda v: v.reshape(1, -1)

    h = feat
    degp3 = None
    for i in range(len(convs)):
        p = convs[i]
        if i == 0:
            aggp, degp = _agg_deg(h, src, dst, zeros, ones)
            degp3 = degp[:, :, None]
        else:
            (aggp,) = _agg(h, src, dst, zeros, ones)
        if i < len(convs) - 1:
            h = _conv_tc(h, aggp, degp3, p["W_self"], p["W_neigh"],
                         row(p["gamma"]), row(p["beta"]))
        else:
            h = _final_tc(h, aggp, degp3, p["W_self"], p["W_neigh"],
                          row(p["gamma"]), row(p["beta"]),
                          c0["W"], row(c0["b"]), row(c0["gamma"]),
                          row(c0["beta"]), c1["W"], row(c1["b"]))
    return h


# speedup vs baseline: 3.2355x; 1.2451x over previous
import functools
import math

import jax
import jax.numpy as jnp
from jax import lax
from jax.experimental import pallas as pl
from jax.experimental.pallas import tpu as pltpu
from jax.experimental.pallas import tpu_sc as plsc

N = 10000
E = 320000
D = 128
OUT = 40
EPS = 1e-5
BNS = 1.0 / math.sqrt(1.0 + EPS)

NC, NS = 2, 16
NW = NC * NS
CB = 128
NCHUNK = E // CB
CPW = NCHUNK // NW
XW = NCHUNK - CPW * NW
BQ = 6
NSUP = CPW // BQ
NPAD = 10240
RPT = NPAD // NS
QCH = RPT // CB


def _make_agg(with_deg: bool):
    out_type = [jax.ShapeDtypeStruct((NC, NPAD, D), jnp.float32)]
    if with_deg:
        out_type.append(jax.ShapeDtypeStruct((NC, NPAD), jnp.float32))

    scratch = [
        pltpu.VMEM((BQ * CB,), jnp.int32),
        pltpu.VMEM((BQ * CB,), jnp.int32),
        pltpu.VMEM((CB,), jnp.int32),
        pltpu.VMEM((CB,), jnp.int32),
        pltpu.VMEM((CB, D), jnp.float32),
        pltpu.VMEM((CB, D), jnp.float32),
        pltpu.VMEM((CB,), jnp.float32),
        pltpu.VMEM((CB,), jnp.float32),
        pltpu.VMEM_SHARED((NPAD, D), jnp.float32),
        pltpu.VMEM_SHARED((NPAD,), jnp.float32),
        pltpu.SemaphoreType.DMA,
    ]

    def body(h_hbm, src_hbm, dst_hbm, z_hbm, one_hbm, *refs):
        if with_deg:
            aggp, degp = refs[0], refs[1]
            rest = refs[2:]
        else:
            aggp, degp = refs[0], None
            rest = refs[1:]
        (sb, db, xs, xd, rowsA, rowsB, ones_v, dv,
         acc_sh, deg_sh, sem_g) = rest

        c = lax.axis_index("c")
        s = lax.axis_index("s")
        w = s * NC + c

        pltpu.sync_copy(z_hbm, rowsA)
        pltpu.sync_copy(one_hbm, ones_v)

        def zero_q(q, carry):
            off = s * RPT + q * CB
            pltpu.sync_copy(rowsA, acc_sh.at[pl.ds(off, CB)])
            if with_deg:
                pltpu.sync_copy(rowsA.at[0], deg_sh.at[pl.ds(off, CB)])
            return carry

        lax.fori_loop(0, QCH, zero_q, 0)
        plsc.subcore_barrier()

        def fire_g(ib, j, rbuf):
            pltpu.async_copy(h_hbm.at[ib.at[pl.ds(j * CB, CB)]], rbuf, sem_g)

        def drain_g(ib, j, rbuf):
            pltpu.make_async_copy(h_hbm.at[ib.at[pl.ds(j * CB, CB)]], rbuf,
                                  sem_g).wait()

        def scat(ib, j, rbuf):
            pltpu.sync_copy(rbuf, acc_sh.at[ib.at[pl.ds(j * CB, CB)]],
                            add=True)
            if with_deg:
                pltpu.sync_copy(ones_v, deg_sh.at[ib.at[pl.ds(j * CB, CB)]],
                                add=True)

        @pl.when(w < XW)
        def _():
            b = (NW * CPW + w) * CB
            pltpu.sync_copy(src_hbm.at[pl.ds(b, CB)], xs)
            pltpu.sync_copy(dst_hbm.at[pl.ds(b, CB)], xd)
            pltpu.async_copy(h_hbm.at[xs], rowsA, sem_g).wait()
            pltpu.sync_copy(rowsA, acc_sh.at[xd], add=True)
            if with_deg:
                pltpu.sync_copy(ones_v, deg_sh.at[xd], add=True)

        base = w * CPW * CB

        def sup(q, carry):
            b0 = base + q * BQ * CB
            pltpu.sync_copy(src_hbm.at[pl.ds(b0, BQ * CB)], sb)
            pltpu.sync_copy(dst_hbm.at[pl.ds(b0, BQ * CB)], db)
            fire_g(sb, 0, rowsA)
            for j in range(BQ):
                rcur = rowsA if j % 2 == 0 else rowsB
                rnxt = rowsB if j % 2 == 0 else rowsA
                drain_g(sb, j, rcur)
                if j + 1 < BQ:
                    fire_g(sb, j + 1, rnxt)
                scat(db, j, rcur)
            return carry

        lax.fori_loop(0, NSUP, sup, 0)

        plsc.subcore_barrier()

        def wb_q(q, carry):
            off = s * RPT + q * CB
            pltpu.sync_copy(acc_sh.at[pl.ds(off, CB)], rowsA)
            pltpu.sync_copy(rowsA, aggp.at[c, pl.ds(off, CB)])
            if with_deg:
                pltpu.sync_copy(deg_sh.at[pl.ds(off, CB)], dv)
                pltpu.sync_copy(dv, degp.at[c, pl.ds(off, CB)])
            return carry

        lax.fori_loop(0, QCH, wb_q, 0)

    mesh = plsc.VectorSubcoreMesh(
        core_axis_name="c", subcore_axis_name="s",
        num_cores=NC, num_subcores=NS)
    return pl.kernel(body, out_type=out_type, mesh=mesh,
                     scratch_types=scratch)


_agg_deg = _make_agg(True)
_agg = _make_agg(False)

BR = 1000
GRID = N // BR


def _conv_body(h_ref, p_ref, d_ref, ws_ref, wn_ref, g_ref, b_ref, o_ref):
    dsum = jnp.maximum(d_ref[0] + d_ref[1], 1.0)
    agg = (p_ref[0] + p_ref[1]) / dsum
    rst = (jnp.dot(h_ref[...], ws_ref[...], preferred_element_type=jnp.float32)
           + jnp.dot(agg, wn_ref[...], preferred_element_type=jnp.float32))
    y = rst * (g_ref[0] * BNS) + b_ref[0]
    o_ref[...] = jnp.maximum(y, 0.0)


def _final_body(h_ref, p_ref, d_ref, ws_ref, wn_ref, g_ref, b_ref,
                w0_ref, b0_ref, g0_ref, be0_ref, w1_ref, b1_ref, o_ref):
    dsum = jnp.maximum(d_ref[0] + d_ref[1], 1.0)
    agg = (p_ref[0] + p_ref[1]) / dsum
    rst = (jnp.dot(h_ref[...], ws_ref[...], preferred_element_type=jnp.float32)
           + jnp.dot(agg, wn_ref[...], preferred_element_type=jnp.float32))
    h3 = jnp.maximum(rst * (g_ref[0] * BNS) + b_ref[0], 0.0)
    t = jnp.dot(h3, w0_ref[...], preferred_element_type=jnp.float32) + b0_ref[0]
    t = jnp.maximum(t * (g0_ref[0] * BNS) + be0_ref[0], 0.0)
    o_ref[...] = (jnp.dot(t, w1_ref[...], preferred_element_type=jnp.float32)
                  + b1_ref[0])


_ROWS = pl.BlockSpec((BR, D), lambda i: (i, 0))
_PART = pl.BlockSpec((NC, BR, D), lambda i: (0, i, 0))
_DEG = pl.BlockSpec((NC, BR, 1), lambda i: (0, i, 0))
_MAT = pl.BlockSpec((D, D), lambda i: (0, 0))
_VEC = pl.BlockSpec((1, D), lambda i: (0, 0))

_conv_tc = pl.pallas_call(
    _conv_body,
    grid=(GRID,),
    in_specs=[_ROWS, _PART, _DEG, _MAT, _MAT, _VEC, _VEC],
    out_specs=_ROWS,
    out_shape=jax.ShapeDtypeStruct((N, D), jnp.float32),
)

_final_tc = pl.pallas_call(
    _final_body,
    grid=(GRID,),
    in_specs=[_ROWS, _PART, _DEG, _MAT, _MAT, _VEC, _VEC,
              _MAT, _VEC, _VEC, _VEC,
              pl.BlockSpec((D, OUT), lambda i: (0, 0)),
              pl.BlockSpec((1, OUT), lambda i: (0, 0))],
    out_specs=pl.BlockSpec((BR, OUT), lambda i: (i, 0)),
    out_shape=jax.ShapeDtypeStruct((N, OUT), jnp.float32),
)


def kernel(feat, params, edge_index):
    src = edge_index[0]
    dst = edge_index[1]
    zeros = jnp.zeros((CB, D), jnp.float32)
    ones = jnp.ones((CB,), jnp.float32)

    convs = params["convs"]
    c0, c1 = params["cls"][0], params["cls"][1]
    row = lambda v: v.reshape(1, -1)

    h = feat
    degp3 = None
    for i in range(len(convs)):
        p = convs[i]
        if i == 0:
            aggp, degp = _agg_deg(h, src, dst, zeros, ones)
            degp3 = degp[:, :, None]
        else:
            (aggp,) = _agg(h, src, dst, zeros, ones)
        if i < len(convs) - 1:
            h = _conv_tc(h, aggp, degp3, p["W_self"], p["W_neigh"],
                         row(p["gamma"]), row(p["beta"]))
        else:
            h = _final_tc(h, aggp, degp3, p["W_self"], p["W_neigh"],
                          row(p["gamma"]), row(p["beta"]),
                          c0["W"], row(c0["b"]), row(c0["gamma"]),
                          row(c0["beta"]), c1["W"], row(c1["b"]))
    return h


# speedup vs baseline: 3.3132x; 1.0240x over previous
import functools
import math

import jax
import jax.numpy as jnp
from jax import lax
from jax.experimental import pallas as pl
from jax.experimental.pallas import tpu as pltpu
from jax.experimental.pallas import tpu_sc as plsc

N = 10000
E = 320000
D = 128
OUT = 40
EPS = 1e-5
BNS = 1.0 / math.sqrt(1.0 + EPS)

NC, NS = 2, 16
NW = NC * NS
CB = 128
NCHUNK = E // CB
CPW = NCHUNK // NW
XW = NCHUNK - CPW * NW
BQ = 6
NSUP = CPW // BQ
NPAD = 10240
RPT = NPAD // NS
QCH = RPT // CB


def _make_agg(with_deg: bool):
    out_type = [jax.ShapeDtypeStruct((NC, NPAD, D), jnp.float32)]
    if with_deg:
        out_type.append(jax.ShapeDtypeStruct((NC, NPAD), jnp.float32))

    scratch = [
        pltpu.VMEM((BQ * CB,), jnp.int32),
        pltpu.VMEM((BQ * CB,), jnp.int32),
        pltpu.VMEM((CB,), jnp.int32),
        pltpu.VMEM((CB,), jnp.int32),
        pltpu.VMEM((CB, D), jnp.float32),
        pltpu.VMEM((CB, D), jnp.float32),
        pltpu.VMEM((CB,), jnp.float32),
        pltpu.VMEM((RPT,), jnp.float32),
        pltpu.VMEM_SHARED((NPAD, D), jnp.float32),
        pltpu.VMEM_SHARED((NPAD,), jnp.float32),
        pltpu.SemaphoreType.DMA,
    ]

    def body(h_hbm, src_hbm, dst_hbm, z_hbm, one_hbm, *refs):
        if with_deg:
            aggp, degp = refs[0], refs[1]
            rest = refs[2:]
        else:
            aggp, degp = refs[0], None
            rest = refs[1:]
        (sb, db, xs, xd, rowsA, rowsB, ones_v, dv,
         acc_sh, deg_sh, sem_g) = rest

        c = lax.axis_index("c")
        s = lax.axis_index("s")
        w = s * NC + c

        pltpu.sync_copy(z_hbm, rowsA)
        pltpu.sync_copy(one_hbm, ones_v)

        def zero_q(q, carry):
            off = s * RPT + q * CB
            pltpu.sync_copy(rowsA, acc_sh.at[pl.ds(off, CB)])
            if with_deg:
                pltpu.sync_copy(rowsA.at[0], deg_sh.at[pl.ds(off, CB)])
            return carry

        lax.fori_loop(0, QCH, zero_q, 0)
        plsc.subcore_barrier()

        def fire_g(ib, j, rbuf):
            pltpu.async_copy(h_hbm.at[ib.at[pl.ds(j * CB, CB)]], rbuf, sem_g)

        def drain_g(ib, j, rbuf):
            pltpu.make_async_copy(h_hbm.at[ib.at[pl.ds(j * CB, CB)]], rbuf,
                                  sem_g).wait()

        def scat(ib, j, rbuf):
            pltpu.sync_copy(rbuf, acc_sh.at[ib.at[pl.ds(j * CB, CB)]],
                            add=True)
            if with_deg:
                pltpu.sync_copy(ones_v, deg_sh.at[ib.at[pl.ds(j * CB, CB)]],
                                add=True)

        @pl.when(w < XW)
        def _():
            b = (NW * CPW + w) * CB
            pltpu.sync_copy(src_hbm.at[pl.ds(b, CB)], xs)
            pltpu.sync_copy(dst_hbm.at[pl.ds(b, CB)], xd)
            pltpu.async_copy(h_hbm.at[xs], rowsA, sem_g).wait()
            pltpu.sync_copy(rowsA, acc_sh.at[xd], add=True)
            if with_deg:
                pltpu.sync_copy(ones_v, deg_sh.at[xd], add=True)

        base = w * CPW * CB

        def sup(q, carry):
            b0 = base + q * BQ * CB
            pltpu.sync_copy(src_hbm.at[pl.ds(b0, BQ * CB)], sb)
            pltpu.sync_copy(dst_hbm.at[pl.ds(b0, BQ * CB)], db)
            fire_g(sb, 0, rowsA)
            for j in range(BQ):
                rcur = rowsA if j % 2 == 0 else rowsB
                rnxt = rowsB if j % 2 == 0 else rowsA
                drain_g(sb, j, rcur)
                if j + 1 < BQ:
                    fire_g(sb, j + 1, rnxt)
                scat(db, j, rcur)
            return carry

        lax.fori_loop(0, NSUP, sup, 0)

        plsc.subcore_barrier()

        if with_deg:
            pltpu.sync_copy(deg_sh.at[pl.ds(s * RPT, RPT)], dv)
            pltpu.sync_copy(dv, degp.at[c, pl.ds(s * RPT, RPT)])

        def wb_q(q, carry):
            off = s * RPT + q * CB
            pltpu.sync_copy(acc_sh.at[pl.ds(off, CB)], rowsA)
            pltpu.sync_copy(rowsA, aggp.at[c, pl.ds(off, CB)])
            return carry

        lax.fori_loop(0, QCH, wb_q, 0)

    mesh = plsc.VectorSubcoreMesh(
        core_axis_name="c", subcore_axis_name="s",
        num_cores=NC, num_subcores=NS)
    return pl.kernel(body, out_type=out_type, mesh=mesh,
                     scratch_types=scratch)


_agg_deg = _make_agg(True)
_agg = _make_agg(False)

BR = 2000
GRID = N // BR


def _conv_body(h_ref, p_ref, d_ref, ws_ref, wn_ref, g_ref, b_ref, o_ref):
    dsum = jnp.maximum(d_ref[0] + d_ref[1], 1.0)
    agg = (p_ref[0] + p_ref[1]) / dsum
    rst = (jnp.dot(h_ref[...], ws_ref[...], preferred_element_type=jnp.float32)
           + jnp.dot(agg, wn_ref[...], preferred_element_type=jnp.float32))
    y = rst * (g_ref[0] * BNS) + b_ref[0]
    o_ref[...] = jnp.maximum(y, 0.0)


def _final_body(h_ref, p_ref, d_ref, ws_ref, wn_ref, g_ref, b_ref,
                w0_ref, b0_ref, g0_ref, be0_ref, w1_ref, b1_ref, o_ref):
    dsum = jnp.maximum(d_ref[0] + d_ref[1], 1.0)
    agg = (p_ref[0] + p_ref[1]) / dsum
    rst = (jnp.dot(h_ref[...], ws_ref[...], preferred_element_type=jnp.float32)
           + jnp.dot(agg, wn_ref[...], preferred_element_type=jnp.float32))
    h3 = jnp.maximum(rst * (g_ref[0] * BNS) + b_ref[0], 0.0)
    t = jnp.dot(h3, w0_ref[...], preferred_element_type=jnp.float32) + b0_ref[0]
    t = jnp.maximum(t * (g0_ref[0] * BNS) + be0_ref[0], 0.0)
    o_ref[...] = (jnp.dot(t, w1_ref[...], preferred_element_type=jnp.float32)
                  + b1_ref[0])


_ROWS = pl.BlockSpec((BR, D), lambda i: (i, 0))
_PART = pl.BlockSpec((NC, BR, D), lambda i: (0, i, 0))
_DEG = pl.BlockSpec((NC, BR, 1), lambda i: (0, i, 0))
_MAT = pl.BlockSpec((D, D), lambda i: (0, 0))
_VEC = pl.BlockSpec((1, D), lambda i: (0, 0))

_conv_tc = pl.pallas_call(
    _conv_body,
    grid=(GRID,),
    in_specs=[_ROWS, _PART, _DEG, _MAT, _MAT, _VEC, _VEC],
    out_specs=_ROWS,
    out_shape=jax.ShapeDtypeStruct((N, D), jnp.float32),
)

_final_tc = pl.pallas_call(
    _final_body,
    grid=(GRID,),
    in_specs=[_ROWS, _PART, _DEG, _MAT, _MAT, _VEC, _VEC,
              _MAT, _VEC, _VEC, _VEC,
              pl.BlockSpec((D, OUT), lambda i: (0, 0)),
              pl.BlockSpec((1, OUT), lambda i: (0, 0))],
    out_specs=pl.BlockSpec((BR, OUT), lambda i: (i, 0)),
    out_shape=jax.ShapeDtypeStruct((N, OUT), jnp.float32),
)


def kernel(feat, params, edge_index):
    src = edge_index[0]
    dst = edge_index[1]
    zeros = jnp.zeros((CB, D), jnp.float32)
    ones = jnp.ones((CB,), jnp.float32)

    convs = params["convs"]
    c0, c1 = params["cls"][0], params["cls"][1]
    row = lambda v: v.reshape(1, -1)

    h = feat
    degp3 = None
    for i in range(len(convs)):
        p = convs[i]
        if i == 0:
            aggp, degp = _agg_deg(h, src, dst, zeros, ones)
            degp3 = degp[:, :, None]
        else:
            (aggp,) = _agg(h, src, dst, zeros, ones)
        if i < len(convs) - 1:
            h = _conv_tc(h, aggp, degp3, p["W_self"], p["W_neigh"],
                         row(p["gamma"]), row(p["beta"]))
        else:
            h = _final_tc(h, aggp, degp3, p["W_self"], p["W_neigh"],
                          row(p["gamma"]), row(p["beta"]),
                          c0["W"], row(c0["b"]), row(c0["gamma"]),
                          row(c0["beta"]), c1["W"], row(c1["b"]))
    return h
